# baseline (speedup 1.0000x reference)
import jax
import jax.numpy as jnp
from jax import lax
from jax.experimental import pallas as pl
from jax.experimental.pallas import tpu as pltpu

B, T, NM, L, NL = 32, 1000, 80, 200, 4
TP = 1024


def _body(ml_ref, mp_ref, mt_ref, go_ref, gt_ref, vm_ref, vmc_ref,
          mel_len_ref, seq_len_ref, a2_ref,
          out_lin, out_post, out_gate, out_guide,
          acc_lin, acc_post, acc_gate, acc_valid, acc_guide, acc_s):
    b = pl.program_id(0)

    @pl.when(b == 0)
    def _init():
        acc_lin[...] = jnp.zeros_like(acc_lin)
        acc_post[...] = jnp.zeros_like(acc_post)
        acc_gate[...] = jnp.zeros_like(acc_gate)
        acc_valid[...] = jnp.zeros_like(acc_valid)
        acc_guide[...] = jnp.zeros_like(acc_guide)
        acc_s[0] = 0.0

    vmc = vmc_ref[0]
    ml = ml_ref[0]
    mp = mp_ref[0]
    mt = mt_ref[0]
    acc_lin[...] += jnp.abs(ml - mt) * vmc
    acc_post[...] += jnp.abs(mp - mt) * vmc

    x = go_ref[0]
    z = gt_ref[0]
    valid = vm_ref[0]
    bce = jnp.maximum(x, 0.0) - x * z + jnp.log(1.0 + jnp.exp(-jnp.abs(x)))
    acc_gate[...] += bce * valid
    acc_valid[...] += valid

    t_i = mel_len_ref[b].astype(jnp.float32)
    l_i = seq_len_ref[b].astype(jnp.float32)
    inv_t = 1.0 / jnp.maximum(t_i, 1.0)
    inv_l = 1.0 / jnp.maximum(l_i, 1.0)
    tcol = lax.broadcasted_iota(jnp.int32, (T, 1), 0).astype(jnp.float32) + 1.0
    lrow = lax.broadcasted_iota(jnp.int32, (1, L), 1).astype(jnp.float32) + 1.0
    tmask = jnp.where(tcol <= t_i, 1.0, 0.0)
    lmask = jnp.where(lrow <= l_i, 1.0, 0.0)
    tn = tcol * inv_t * tmask
    ln = lrow * inv_l
    diff = tn - ln
    w = (1.0 - jnp.exp(-12.5 * (diff * diff))) * tmask * lmask
    d = a2_ref[0]
    acc_guide[...] += (d[0] + d[1]) * w
    tc = jnp.minimum(jnp.maximum(mel_len_ref[b], 0), T).astype(jnp.float32)
    lc = jnp.minimum(jnp.maximum(seq_len_ref[b], 0), L).astype(jnp.float32)
    acc_s[0] += tc * lc

    @pl.when(b == B - 1)
    def _fin():
        vcount = jnp.maximum(jnp.sum(acc_valid[...]), 1.0)
        out_lin[0, 0] = jnp.sum(acc_lin[...]) / (vcount * NM)
        out_post[0, 0] = jnp.sum(acc_post[...]) / (vcount * NM)
        out_gate[0, 0] = jnp.sum(acc_gate[...]) / vcount
        den = jnp.maximum(2.0 * acc_s[0], 1.0)
        out_guide[0, 0] = 10.0 * jnp.sum(acc_guide[...]) / den


def kernel(mel_linear, mel_post, gate_out, mel_target, gate_target, mel_mask,
           mel_len, seq_len, alignments2):
    valid = 1.0 - mel_mask.astype(jnp.float32)
    pad = ((0, 0), (0, TP - T))
    go_p = jnp.pad(gate_out, pad).reshape(B, 8, 128)
    gt_p = jnp.pad(gate_target, pad).reshape(B, 8, 128)
    vm_p = jnp.pad(valid, pad).reshape(B, 8, 128)
    vm_c = valid.reshape(B, T, 1)
    scalar_shape = jax.ShapeDtypeStruct((1, 1), jnp.float32)
    smem_scalar = pl.BlockSpec((1, 1), lambda b: (0, 0), memory_space=pltpu.SMEM)
    outs = pl.pallas_call(
        _body,
        grid=(B,),
        in_specs=[
            pl.BlockSpec((1, T, NM), lambda b: (b, 0, 0)),
            pl.BlockSpec((1, T, NM), lambda b: (b, 0, 0)),
            pl.BlockSpec((1, T, NM), lambda b: (b, 0, 0)),
            pl.BlockSpec((1, 8, 128), lambda b: (b, 0, 0)),
            pl.BlockSpec((1, 8, 128), lambda b: (b, 0, 0)),
            pl.BlockSpec((1, 8, 128), lambda b: (b, 0, 0)),
            pl.BlockSpec((1, T, 1), lambda b: (b, 0, 0)),
            pl.BlockSpec(memory_space=pltpu.SMEM),
            pl.BlockSpec(memory_space=pltpu.SMEM),
            pl.BlockSpec((1, 2, T, L), lambda b: (b, 1, 0, 0)),
        ],
        out_specs=[smem_scalar] * 4,
        out_shape=[scalar_shape] * 4,
        scratch_shapes=[
            pltpu.VMEM((T, NM), jnp.float32),
            pltpu.VMEM((T, NM), jnp.float32),
            pltpu.VMEM((8, 128), jnp.float32),
            pltpu.VMEM((8, 128), jnp.float32),
            pltpu.VMEM((T, L), jnp.float32),
            pltpu.SMEM((1,), jnp.float32),
        ],
    )(mel_linear, mel_post, mel_target, go_p, gt_p, vm_p, vm_c,
      mel_len.astype(jnp.int32), seq_len.astype(jnp.int32), alignments2)
    return tuple(o[0, 0] for o in outs)
